# baseline (speedup 1.0000x reference)
import functools

import jax
import jax.numpy as jnp
from jax import lax
from jax.experimental import pallas as pl
from jax.experimental.pallas import tpu as pltpu
from jax.experimental.pallas import tpu_sc as plsc

N = 10000
E = 320000
C = 128
CH = C // 2
NC = 2
NS = 16
L = 16

NW = NC * NS
PT = E // NW
K = 80
CHUNKS = PT // K
NPAD = 10240
TROW = NPAD // NS
NB = 10
BN = NPAD // NB


def _interleave16(a, b):
    return jnp.stack([a, b], axis=2).reshape(a.shape[0], 2 * a.shape[1])


def _pre_body(x_ref, w_ref, as_ref, ad_ref, h0_ref, h1_ref, a2_ref, mx_ref):
    h = jnp.dot(x_ref[...], w_ref[...], preferred_element_type=jnp.float32)
    hb = h.astype(jnp.bfloat16)
    h0_ref[...] = jnp.concatenate(
        [_interleave16(hb[:, 0:16], hb[:, 16:32]),
         _interleave16(hb[:, 32:48], hb[:, 48:64])], axis=1)
    h1_ref[...] = jnp.concatenate(
        [_interleave16(hb[:, 64:80], hb[:, 80:96]),
         _interleave16(hb[:, 96:112], hb[:, 112:128])], axis=1)
    s = jnp.sum(h * as_ref[...], axis=1, keepdims=True)
    d = jnp.sum(h * ad_ref[...], axis=1, keepdims=True)
    a2_ref[...] = jnp.concatenate([s, d], axis=1)

    @pl.when(pl.program_id(0) == 0)
    def _():
        mx_ref[...] = jnp.full((2, C), -3.0e38, jnp.float32)

    blk = jnp.concatenate([jnp.full((1, C), jnp.max(s), jnp.float32),
                           jnp.full((1, C), jnp.max(d), jnp.float32)], axis=0)
    mx_ref[...] = jnp.maximum(mx_ref[...], blk)


def _pre(x, W, att_s, att_d):
    return pl.pallas_call(
        _pre_body,
        grid=(10,),
        in_specs=[
            pl.BlockSpec((N // 10, C), lambda i: (i, 0)),
            pl.BlockSpec((C, C), lambda i: (0, 0)),
            pl.BlockSpec((1, C), lambda i: (0, 0)),
            pl.BlockSpec((1, C), lambda i: (0, 0)),
        ],
        out_specs=[
            pl.BlockSpec((N // 10, CH), lambda i: (i, 0)),
            pl.BlockSpec((N // 10, CH), lambda i: (i, 0)),
            pl.BlockSpec((N // 10, 2), lambda i: (i, 0)),
            pl.BlockSpec((2, C), lambda i: (0, 0)),
        ],
        out_shape=[
            jax.ShapeDtypeStruct((N, CH), jnp.bfloat16),
            jax.ShapeDtypeStruct((N, CH), jnp.bfloat16),
            jax.ShapeDtypeStruct((N, 2), jnp.float32),
            jax.ShapeDtypeStruct((2, C), jnp.float32),
        ],
    )(x, W, att_s, att_d)


def _post_body(p_ref, den_ref, b_ref, g_ref, bb_ref, y_ref):
    num = p_ref[0] + p_ref[1]
    cols = jnp.concatenate([num[0], num[1]], axis=1)
    den = jnp.sum(den_ref[...], axis=1)
    srow = cols / (den[:, None] + 1e-16) + b_ref[...]
    mu = jnp.mean(srow, axis=1, keepdims=True)
    var = jnp.mean((srow - mu) ** 2, axis=1, keepdims=True)
    yn = (srow - mu) * lax.rsqrt(var + 1e-5) * g_ref[...] + bb_ref[...]
    y_ref[...] = jnp.where(yn > 0, yn, jnp.exp(yn) - 1.0)


BR = N // 10


def _post(p, dent, bias, g, b):
    return pl.pallas_call(
        _post_body,
        grid=(10,),
        in_specs=[
            pl.BlockSpec((NC, 2, BR, CH), lambda i: (0, 0, i, 0)),
            pl.BlockSpec((BR, NW), lambda i: (i, 0)),
            pl.BlockSpec((1, C), lambda i: (0, 0)),
            pl.BlockSpec((1, C), lambda i: (0, 0)),
            pl.BlockSpec((1, C), lambda i: (0, 0)),
        ],
        out_specs=pl.BlockSpec((BR, C), lambda i: (i, 0)),
        out_shape=jax.ShapeDtypeStruct((N, C), jnp.float32),
    )(p, dent, bias, g, b)


@functools.partial(
    pl.kernel,
    out_type=[
        jax.ShapeDtypeStruct((NC, 2, NPAD, CH), jnp.float32),
        jax.ShapeDtypeStruct((NC, NS, NPAD), jnp.float32),
    ],
    mesh=plsc.VectorSubcoreMesh(core_axis_name="c", subcore_axis_name="s"),
    compiler_params=pltpu.CompilerParams(needs_layout_passes=False,
                                         use_tc_tiling_on_sc=False),
    scratch_types=[
        pltpu.VMEM((N,), jnp.float32),
        pltpu.VMEM((N,), jnp.float32),
        pltpu.VMEM((PT,), jnp.float32),
        pltpu.VMEM((NPAD,), jnp.float32),
        pltpu.VMEM((CHUNKS, K), jnp.int32),
        pltpu.VMEM((CHUNKS, K), jnp.int32),
        pltpu.VMEM((2, K, CH), jnp.bfloat16),
        pltpu.VMEM((2, K, CH), jnp.float32),
        pltpu.VMEM((C,), jnp.float32),
        pltpu.SemaphoreType.DMA,
        pltpu.SemaphoreType.DMA,
        pltpu.SemaphoreType.DMA,
        pltpu.SemaphoreType.DMA,
        pltpu.VMEM_SHARED((NPAD, CH), jnp.float32),
    ],
)
def _sc_edge(h0_hbm, h1_hbm, asrc_hbm, adst_hbm, msh_hbm, e2_hbm,
             outp_hbm, denp_hbm,
             asrc_v, adst_v, e_v, den_v, src_v, dst_v, rowsb_v, rowsf_v,
             msh_v, gsem0, gsem1, ssem0, ssem1, sh_out):
    c = lax.axis_index("c")
    s = lax.axis_index("s")
    wid = c * NS + s

    zf = jnp.zeros((L,), jnp.float32)

    pltpu.sync_copy(asrc_hbm, asrc_v)
    pltpu.sync_copy(adst_hbm, adst_v)
    pltpu.sync_copy(msh_hbm, msh_v)
    pltpu.sync_copy(e2_hbm.at[0, wid], src_v)
    pltpu.sync_copy(e2_hbm.at[1, wid], dst_v)

    def _zrow(i, _):
        for v in range(CH // L):
            rowsf_v[0, i, pl.ds(v * L, L)] = zf
        return 0

    lax.fori_loop(0, K, _zrow, 0)

    def _zden(i, _):
        den_v[pl.ds(i * L, L)] = zf
        return 0

    lax.fori_loop(0, NPAD // L, _zden, 0)

    m_shift = msh_v[pl.ds(0, L)]
    row_base = pl.multiple_of(s * TROW, 8)

    for half in range(2):
        h_hbm = (h0_hbm, h1_hbm)[half]

        for k in range(TROW // K):
            pltpu.sync_copy(rowsf_v.at[0],
                            sh_out.at[pl.ds(row_base + k * K, K)])
        plsc.subcore_barrier()

        def _gather(i, gb, h_hbm=h_hbm):
            sem = (gsem0, gsem1)[gb]
            return pltpu.make_async_copy(h_hbm.at[src_v.at[i]],
                                         rowsb_v.at[gb], sem)

        def _scat_start(i, fb):
            pltpu.async_copy(rowsf_v.at[fb], sh_out.at[dst_v.at[i]],
                             (ssem0, ssem1)[fb], add=True)

        def _scat_wait(i, fb):
            pltpu.make_async_copy(rowsf_v.at[fb], sh_out.at[dst_v.at[i]],
                                  (ssem0, ssem1)[fb]).wait()

        def _chunk(i, gb, fb, half=half):
            if half == 0:
                for g in range(K // L):
                    si = src_v[i, pl.ds(g * L, L)]
                    di = dst_v[i, pl.ds(g * L, L)]
                    al = (plsc.load_gather(asrc_v, [si])
                          + plsc.load_gather(adst_v, [di]))
                    al = jnp.where(al >= 0, al, 0.2 * al)
                    e = jnp.exp(al - m_shift)
                    e_v[pl.ds(i * K + g * L, L)] = e
                    plsc.addupdate_scatter(den_v, [di], e)
            for g in range(K // L):
                ev = e_v[pl.ds(i * K + g * L, L)]
                for j in range(L):
                    aj = jnp.broadcast_to(ev[j], (L,))
                    r = g * L + j
                    for v in range(CH // 32):
                        x = rowsb_v[gb, r, pl.ds(32 * v, 32)]
                        u, w = plsc.unpack(
                            x, format=plsc.PackFormat.INTERLEAVED)
                        rowsf_v[fb, r, pl.ds(32 * v, L)] = u * aj
                        rowsf_v[fb, r, pl.ds(32 * v + L, L)] = w * aj

        def _step(i, b, first, ahead=True):
            if not first:
                _scat_wait(i - 2, b)
            if ahead:
                _gather(i + 1, 1 - b).start()
            _gather(i, b).wait()
            _chunk(i, b, b)
            _scat_start(i, b)

        _gather(0, 0).start()
        _step(0, 0, True)
        _step(1, 1, True)
        _step(2, 0, False)

        def _two(t, _):
            i0 = 3 + t * 2
            _step(i0, 1, False)
            _step(i0 + 1, 0, False)
            return 0

        lax.fori_loop(0, (CHUNKS - 5) // 2, _two, 0)
        for i in range(CHUNKS - 2, CHUNKS):
            _step(i, i % 2, False, ahead=(i + 1 < CHUNKS))
        _scat_wait(CHUNKS - 2, (CHUNKS - 2) % 2)
        _scat_wait(CHUNKS - 1, (CHUNKS - 1) % 2)

        plsc.subcore_barrier()
        pltpu.sync_copy(sh_out.at[pl.ds(row_base, TROW)],
                        outp_hbm.at[c, half, pl.ds(row_base, TROW), :])

        def _zr2(i, _):
            for v in range(CH // L):
                rowsf_v[0, i, pl.ds(v * L, L)] = zf
            return 0

        if half == 0:
            lax.fori_loop(0, K, _zr2, 0)

    pltpu.sync_copy(den_v, denp_hbm.at[c, s])


def kernel(x, edge_index, W, att_src, att_dst, bias, ln_gamma, ln_beta):
    h0, h1, a2, mx = _pre(x, W, att_src.reshape(1, C), att_dst.reshape(1, C))
    asrc = a2[:, 0]
    adst = a2[:, 1]
    m0 = mx[0] + mx[1]
    msh = jnp.where(m0 >= 0, m0, 0.2 * m0)
    e2 = edge_index.reshape(2, NW, CHUNKS, K)
    outp, denp = _sc_edge(h0, h1, asrc, adst, msh, e2)
    dent = denp.reshape(NW, NPAD).transpose(1, 0)
    return _post(outp, dent, bias.reshape(1, C), ln_gamma.reshape(1, C),
                 ln_beta.reshape(1, C))


# speedup vs baseline: 1.4395x; 1.4395x over previous
import functools

import jax
import jax.numpy as jnp
from jax import lax
from jax.experimental import pallas as pl
from jax.experimental.pallas import tpu as pltpu
from jax.experimental.pallas import tpu_sc as plsc

N = 10000
E = 320000
C = 128
CH = C // 2
NC = 2
NS = 16
L = 16

NW = NC * NS
PT = E // NW
K = 80
CHUNKS = PT // K
NPAD = 10240
TROW = NPAD // NS
NB = 10
BN = NPAD // NB


_PERM = [32 * g + (k // 2 if k % 2 == 0 else 16 + k // 2)
         for g in range(4) for k in range(32)]


def _pre_body(x_ref, w_ref, as_ref, ad_ref, h0_ref, h1_ref, a2_ref, mx_ref):
    h = jnp.dot(x_ref[...], w_ref[...], preferred_element_type=jnp.float32)
    hb = h.astype(jnp.bfloat16)
    h0_ref[...] = hb[:, :CH]
    h1_ref[...] = hb[:, CH:]
    s = jnp.sum(h * as_ref[...], axis=1, keepdims=True)
    d = jnp.sum(h * ad_ref[...], axis=1, keepdims=True)
    a2_ref[...] = jnp.concatenate([s, d], axis=1)

    @pl.when(pl.program_id(0) == 0)
    def _():
        mx_ref[...] = jnp.full((2, C), -3.0e38, jnp.float32)

    blk = jnp.concatenate([jnp.full((1, C), jnp.max(s), jnp.float32),
                           jnp.full((1, C), jnp.max(d), jnp.float32)], axis=0)
    mx_ref[...] = jnp.maximum(mx_ref[...], blk)


def _pre(x, W, att_s, att_d):
    return pl.pallas_call(
        _pre_body,
        grid=(10,),
        in_specs=[
            pl.BlockSpec((N // 10, C), lambda i: (i, 0)),
            pl.BlockSpec((C, C), lambda i: (0, 0)),
            pl.BlockSpec((1, C), lambda i: (0, 0)),
            pl.BlockSpec((1, C), lambda i: (0, 0)),
        ],
        out_specs=[
            pl.BlockSpec((N // 10, CH), lambda i: (i, 0)),
            pl.BlockSpec((N // 10, CH), lambda i: (i, 0)),
            pl.BlockSpec((N // 10, 2), lambda i: (i, 0)),
            pl.BlockSpec((2, C), lambda i: (0, 0)),
        ],
        out_shape=[
            jax.ShapeDtypeStruct((N, CH), jnp.bfloat16),
            jax.ShapeDtypeStruct((N, CH), jnp.bfloat16),
            jax.ShapeDtypeStruct((N, 2), jnp.float32),
            jax.ShapeDtypeStruct((2, C), jnp.float32),
        ],
    )(x, W, att_s, att_d)


def _post_body(p_ref, den_ref, b_ref, g_ref, bb_ref, y_ref):
    num = p_ref[0] + p_ref[1]
    cols = jnp.concatenate([num[0], num[1]], axis=1)
    den = jnp.sum(den_ref[...], axis=1)
    srow = cols / (den[:, None] + 1e-16) + b_ref[...]
    mu = jnp.mean(srow, axis=1, keepdims=True)
    var = jnp.mean((srow - mu) ** 2, axis=1, keepdims=True)
    yn = (srow - mu) * lax.rsqrt(var + 1e-5) * g_ref[...] + bb_ref[...]
    y_ref[...] = jnp.where(yn > 0, yn, jnp.exp(yn) - 1.0)


BR = N // 10


def _post(p, dent, bias, g, b):
    return pl.pallas_call(
        _post_body,
        grid=(10,),
        in_specs=[
            pl.BlockSpec((NC, 2, BR, CH), lambda i: (0, 0, i, 0)),
            pl.BlockSpec((BR, NW), lambda i: (i, 0)),
            pl.BlockSpec((1, C), lambda i: (0, 0)),
            pl.BlockSpec((1, C), lambda i: (0, 0)),
            pl.BlockSpec((1, C), lambda i: (0, 0)),
        ],
        out_specs=pl.BlockSpec((BR, C), lambda i: (i, 0)),
        out_shape=jax.ShapeDtypeStruct((N, C), jnp.float32),
    )(p, dent, bias, g, b)


@functools.partial(
    pl.kernel,
    out_type=[
        jax.ShapeDtypeStruct((NC, 2, NPAD, CH), jnp.float32),
        jax.ShapeDtypeStruct((NC, NS, NPAD), jnp.float32),
    ],
    mesh=plsc.VectorSubcoreMesh(core_axis_name="c", subcore_axis_name="s"),
    compiler_params=pltpu.CompilerParams(needs_layout_passes=False,
                                         use_tc_tiling_on_sc=False),
    scratch_types=[
        pltpu.VMEM((N,), jnp.float32),
        pltpu.VMEM((N,), jnp.float32),
        pltpu.VMEM((PT,), jnp.float32),
        pltpu.VMEM((NPAD,), jnp.float32),
        pltpu.VMEM((CHUNKS, K), jnp.int32),
        pltpu.VMEM((CHUNKS, K), jnp.int32),
        pltpu.VMEM((2, K, CH), jnp.bfloat16),
        pltpu.VMEM((2, K, CH), jnp.float32),
        pltpu.VMEM((C,), jnp.float32),
        pltpu.SemaphoreType.DMA,
        pltpu.SemaphoreType.DMA,
        pltpu.SemaphoreType.DMA,
        pltpu.SemaphoreType.DMA,
        pltpu.VMEM_SHARED((NPAD, CH), jnp.float32),
    ],
)
def _sc_edge(h0_hbm, h1_hbm, asrc_hbm, adst_hbm, msh_hbm, e2_hbm,
             outp_hbm, denp_hbm,
             asrc_v, adst_v, e_v, den_v, src_v, dst_v, rowsb_v, rowsf_v,
             msh_v, gsem0, gsem1, ssem0, ssem1, sh_out):
    c = lax.axis_index("c")
    s = lax.axis_index("s")
    wid = c * NS + s

    zf = jnp.zeros((L,), jnp.float32)

    pltpu.sync_copy(asrc_hbm, asrc_v)
    pltpu.sync_copy(adst_hbm, adst_v)
    pltpu.sync_copy(msh_hbm, msh_v)
    pltpu.sync_copy(e2_hbm.at[0, wid], src_v)
    pltpu.sync_copy(e2_hbm.at[1, wid], dst_v)

    def _zrow(i, _):
        for v in range(CH // L):
            rowsf_v[0, i, pl.ds(v * L, L)] = zf
        return 0

    lax.fori_loop(0, K, _zrow, 0)

    def _zden(i, _):
        den_v[pl.ds(i * L, L)] = zf
        return 0

    lax.fori_loop(0, NPAD // L, _zden, 0)

    m_shift = msh_v[pl.ds(0, L)]
    row_base = pl.multiple_of(s * TROW, 8)

    for half in range(2):
        h_hbm = (h0_hbm, h1_hbm)[half]

        for k in range(TROW // K):
            pltpu.sync_copy(rowsf_v.at[0],
                            sh_out.at[pl.ds(row_base + k * K, K)])
        plsc.subcore_barrier()

        def _gather(i, gb, h_hbm=h_hbm):
            sem = (gsem0, gsem1)[gb]
            return pltpu.make_async_copy(h_hbm.at[src_v.at[i]],
                                         rowsb_v.at[gb], sem)

        def _scat_start(i, fb):
            pltpu.async_copy(rowsf_v.at[fb], sh_out.at[dst_v.at[i]],
                             (ssem0, ssem1)[fb], add=True)

        def _scat_wait(i, fb):
            pltpu.make_async_copy(rowsf_v.at[fb], sh_out.at[dst_v.at[i]],
                                  (ssem0, ssem1)[fb]).wait()

        def _chunk(i, gb, fb, half=half):
            if half == 0:
                for g in range(K // L):
                    si = src_v[i, pl.ds(g * L, L)]
                    di = dst_v[i, pl.ds(g * L, L)]
                    al = (plsc.load_gather(asrc_v, [si])
                          + plsc.load_gather(adst_v, [di]))
                    al = jnp.where(al >= 0, al, 0.2 * al)
                    e = jnp.exp(al - m_shift)
                    e_v[pl.ds(i * K + g * L, L)] = e
                    plsc.addupdate_scatter(den_v, [di], e)
            for g in range(K // L):
                ev = e_v[pl.ds(i * K + g * L, L)]
                for j in range(L):
                    aj = jnp.broadcast_to(ev[j], (L,))
                    r = g * L + j
                    for v in range(CH // 32):
                        x = rowsb_v[gb, r, pl.ds(32 * v, 32)]
                        u, w = plsc.unpack(
                            x, format=plsc.PackFormat.INTERLEAVED)
                        rowsf_v[fb, r, pl.ds(32 * v, L)] = u * aj
                        rowsf_v[fb, r, pl.ds(32 * v + L, L)] = w * aj

        def _step(i, b, first, ahead=True):
            if not first:
                _scat_wait(i - 2, b)
            if ahead:
                _gather(i + 1, 1 - b).start()
            _gather(i, b).wait()
            _chunk(i, b, b)
            _scat_start(i, b)

        _gather(0, 0).start()
        _step(0, 0, True)
        _step(1, 1, True)
        _step(2, 0, False)

        def _two(t, _):
            i0 = 3 + t * 2
            _step(i0, 1, False)
            _step(i0 + 1, 0, False)
            return 0

        lax.fori_loop(0, (CHUNKS - 5) // 2, _two, 0)
        for i in range(CHUNKS - 2, CHUNKS):
            _step(i, i % 2, False, ahead=(i + 1 < CHUNKS))
        _scat_wait(CHUNKS - 2, (CHUNKS - 2) % 2)
        _scat_wait(CHUNKS - 1, (CHUNKS - 1) % 2)

        plsc.subcore_barrier()
        pltpu.sync_copy(sh_out.at[pl.ds(row_base, TROW)],
                        outp_hbm.at[c, half, pl.ds(row_base, TROW), :])

        def _zr2(i, _):
            for v in range(CH // L):
                rowsf_v[0, i, pl.ds(v * L, L)] = zf
            return 0

        if half == 0:
            lax.fori_loop(0, K, _zr2, 0)

    pltpu.sync_copy(den_v, denp_hbm.at[c, s])


def kernel(x, edge_index, W, att_src, att_dst, bias, ln_gamma, ln_beta):
    perm = jnp.array(_PERM, dtype=jnp.int32)
    Wp = W[:, perm]
    asp = att_src.reshape(1, C)[:, perm]
    adp = att_dst.reshape(1, C)[:, perm]
    h0, h1, a2, mx = _pre(x, Wp, asp, adp)
    asrc = a2[:, 0]
    adst = a2[:, 1]
    m0 = mx[0] + mx[1]
    msh = jnp.where(m0 >= 0, m0, 0.2 * m0)
    e2 = edge_index.reshape(2, NW, CHUNKS, K)
    outp, denp = _sc_edge(h0, h1, asrc, adst, msh, e2)
    dent = denp.reshape(NW, NPAD).transpose(1, 0)
    return _post(outp, dent, bias.reshape(1, C), ln_gamma.reshape(1, C),
                 ln_beta.reshape(1, C))


# speedup vs baseline: 1.4812x; 1.0290x over previous
import functools

import jax
import jax.numpy as jnp
from jax import lax
from jax.experimental import pallas as pl
from jax.experimental.pallas import tpu as pltpu
from jax.experimental.pallas import tpu_sc as plsc

N = 10000
E = 320000
C = 128
CH = C // 2
NC = 2
NS = 16
L = 16

NW = NC * NS
PT = E // NW
K = 80
CHUNKS = PT // K
NPAD = 10240
TROW = NPAD // NS
NB = 10
BN = NPAD // NB


_PERM = [32 * g + (k // 2 if k % 2 == 0 else 16 + k // 2)
         for g in range(4) for k in range(32)]


def _pre_body(x_ref, w_ref, as_ref, ad_ref, h0_ref, h1_ref, a2_ref, mx_ref):
    h = jnp.dot(x_ref[...], w_ref[...], preferred_element_type=jnp.float32)
    hb = h.astype(jnp.bfloat16)
    h0_ref[...] = hb[:, :CH]
    h1_ref[...] = hb[:, CH:]
    s = jnp.sum(h * as_ref[...], axis=1, keepdims=True)
    d = jnp.sum(h * ad_ref[...], axis=1, keepdims=True)
    a2_ref[...] = jnp.concatenate([s, d], axis=1)

    @pl.when(pl.program_id(0) == 0)
    def _():
        mx_ref[...] = jnp.full((2, C), -3.0e38, jnp.float32)

    blk = jnp.concatenate([jnp.full((1, C), jnp.max(s), jnp.float32),
                           jnp.full((1, C), jnp.max(d), jnp.float32)], axis=0)
    mx_ref[...] = jnp.maximum(mx_ref[...], blk)


def _pre(x, W, att_s, att_d):
    return pl.pallas_call(
        _pre_body,
        grid=(10,),
        in_specs=[
            pl.BlockSpec((N // 10, C), lambda i: (i, 0)),
            pl.BlockSpec((C, C), lambda i: (0, 0)),
            pl.BlockSpec((1, C), lambda i: (0, 0)),
            pl.BlockSpec((1, C), lambda i: (0, 0)),
        ],
        out_specs=[
            pl.BlockSpec((N // 10, CH), lambda i: (i, 0)),
            pl.BlockSpec((N // 10, CH), lambda i: (i, 0)),
            pl.BlockSpec((N // 10, 2), lambda i: (i, 0)),
            pl.BlockSpec((2, C), lambda i: (0, 0)),
        ],
        out_shape=[
            jax.ShapeDtypeStruct((N, CH), jnp.bfloat16),
            jax.ShapeDtypeStruct((N, CH), jnp.bfloat16),
            jax.ShapeDtypeStruct((N, 2), jnp.float32),
            jax.ShapeDtypeStruct((2, C), jnp.float32),
        ],
    )(x, W, att_s, att_d)


def _post_body(p_ref, den_ref, b_ref, g_ref, bb_ref, y_ref):
    num = p_ref[0] + p_ref[1]
    cols = jnp.concatenate([num[0], num[1]], axis=1)
    den = jnp.sum(den_ref[...], axis=1)
    srow = cols / (den[:, None] + 1e-16) + b_ref[...]
    mu = jnp.mean(srow, axis=1, keepdims=True)
    var = jnp.mean((srow - mu) ** 2, axis=1, keepdims=True)
    yn = (srow - mu) * lax.rsqrt(var + 1e-5) * g_ref[...] + bb_ref[...]
    y_ref[...] = jnp.where(yn > 0, yn, jnp.exp(yn) - 1.0)


BR = N // 10


def _post(p, dent, bias, g, b):
    return pl.pallas_call(
        _post_body,
        grid=(10,),
        in_specs=[
            pl.BlockSpec((NC, 2, BR, CH), lambda i: (0, 0, i, 0)),
            pl.BlockSpec((BR, NW), lambda i: (i, 0)),
            pl.BlockSpec((1, C), lambda i: (0, 0)),
            pl.BlockSpec((1, C), lambda i: (0, 0)),
            pl.BlockSpec((1, C), lambda i: (0, 0)),
        ],
        out_specs=pl.BlockSpec((BR, C), lambda i: (i, 0)),
        out_shape=jax.ShapeDtypeStruct((N, C), jnp.float32),
    )(p, dent, bias, g, b)


@functools.partial(
    pl.kernel,
    out_type=[
        jax.ShapeDtypeStruct((NC, 2, NPAD, CH), jnp.float32),
        jax.ShapeDtypeStruct((NC, NS, NPAD), jnp.float32),
    ],
    mesh=plsc.VectorSubcoreMesh(core_axis_name="c", subcore_axis_name="s"),
    compiler_params=pltpu.CompilerParams(needs_layout_passes=False,
                                         use_tc_tiling_on_sc=False),
    scratch_types=[
        pltpu.VMEM((N,), jnp.float32),
        pltpu.VMEM((N,), jnp.float32),
        pltpu.VMEM((PT,), jnp.float32),
        pltpu.VMEM((NPAD,), jnp.float32),
        pltpu.VMEM((CHUNKS, K), jnp.int32),
        pltpu.VMEM((CHUNKS, K), jnp.int32),
        pltpu.VMEM((2, K, CH), jnp.bfloat16),
        pltpu.VMEM((2, K, CH), jnp.float32),
        pltpu.VMEM((C,), jnp.float32),
        pltpu.SemaphoreType.DMA,
        pltpu.SemaphoreType.DMA,
        pltpu.SemaphoreType.DMA,
        pltpu.SemaphoreType.DMA,
        pltpu.VMEM_SHARED((NPAD, CH), jnp.float32),
    ],
)
def _sc_edge(h0_hbm, h1_hbm, asrc_hbm, adst_hbm, msh_hbm, e2_hbm,
             outp_hbm, denp_hbm,
             asrc_v, adst_v, e_v, den_v, src_v, dst_v, rowsb_v, rowsf_v,
             msh_v, gsem0, gsem1, ssem0, ssem1, sh_out):
    c = lax.axis_index("c")
    s = lax.axis_index("s")
    wid = c * NS + s

    zf = jnp.zeros((L,), jnp.float32)

    pltpu.sync_copy(asrc_hbm, asrc_v)
    pltpu.sync_copy(adst_hbm, adst_v)
    pltpu.sync_copy(msh_hbm, msh_v)
    pltpu.sync_copy(e2_hbm.at[0, wid], src_v)
    pltpu.sync_copy(e2_hbm.at[1, wid], dst_v)

    def _zrow(i, _):
        for v in range(CH // L):
            rowsf_v[0, i, pl.ds(v * L, L)] = zf
        return 0

    lax.fori_loop(0, K, _zrow, 0)

    def _zden(i, _):
        den_v[pl.ds(i * L, L)] = zf
        return 0

    lax.fori_loop(0, NPAD // L, _zden, 0)

    m_shift = msh_v[pl.ds(0, L)]
    row_base = pl.multiple_of(s * TROW, 8)

    for half in range(2):
        h_hbm = (h0_hbm, h1_hbm)[half]

        for k in range(TROW // K):
            pltpu.sync_copy(rowsf_v.at[0],
                            sh_out.at[pl.ds(row_base + k * K, K)])
        plsc.subcore_barrier()

        def _gather(i, gb, h_hbm=h_hbm):
            sem = (gsem0, gsem1)[gb]
            return pltpu.make_async_copy(h_hbm.at[src_v.at[i]],
                                         rowsb_v.at[gb], sem)

        def _scat_start(i, fb):
            pltpu.async_copy(rowsf_v.at[fb], sh_out.at[dst_v.at[i]],
                             (ssem0, ssem1)[fb], add=True)

        def _scat_wait(i, fb):
            pltpu.make_async_copy(rowsf_v.at[fb], sh_out.at[dst_v.at[i]],
                                  (ssem0, ssem1)[fb]).wait()

        def _att(i):
            for g in range(K // L):
                si = src_v[i, pl.ds(g * L, L)]
                di = dst_v[i, pl.ds(g * L, L)]
                al = (plsc.load_gather(asrc_v, [si])
                      + plsc.load_gather(adst_v, [di]))
                al = jnp.where(al >= 0, al, 0.2 * al)
                e = jnp.exp(al - m_shift)
                e_v[pl.ds(i * K + g * L, L)] = e
                plsc.addupdate_scatter(den_v, [di], e)

        def _scale(i, gb, fb):
            for g in range(K // L):
                ev = e_v[pl.ds(i * K + g * L, L)]
                for j in range(L):
                    aj = jnp.broadcast_to(ev[j], (L,))
                    r = g * L + j
                    for v in range(CH // 32):
                        x = rowsb_v[gb, r, pl.ds(32 * v, 32)]
                        u, w = plsc.unpack(
                            x, format=plsc.PackFormat.INTERLEAVED)
                        rowsf_v[fb, r, pl.ds(32 * v, L)] = u * aj
                        rowsf_v[fb, r, pl.ds(32 * v + L, L)] = w * aj

        def _step(i, b, first, ahead=True, half=half):
            if not first:
                _scat_wait(i - 2, b)
            if ahead:
                _gather(i + 1, 1 - b).start()
            if half == 0:
                _att(i)
            _gather(i, b).wait()
            _scale(i, b, b)
            _scat_start(i, b)

        _gather(0, 0).start()
        _step(0, 0, True)
        _step(1, 1, True)
        _step(2, 0, False)

        def _two(t, _):
            i0 = 3 + t * 2
            _step(i0, 1, False)
            _step(i0 + 1, 0, False)
            return 0

        lax.fori_loop(0, (CHUNKS - 5) // 2, _two, 0)
        for i in range(CHUNKS - 2, CHUNKS):
            _step(i, i % 2, False, ahead=(i + 1 < CHUNKS))
        _scat_wait(CHUNKS - 2, (CHUNKS - 2) % 2)
        _scat_wait(CHUNKS - 1, (CHUNKS - 1) % 2)

        plsc.subcore_barrier()
        pltpu.sync_copy(sh_out.at[pl.ds(row_base, TROW)],
                        outp_hbm.at[c, half, pl.ds(row_base, TROW), :])

        def _zr2(i, _):
            for v in range(CH // L):
                rowsf_v[0, i, pl.ds(v * L, L)] = zf
            return 0

        if half == 0:
            lax.fori_loop(0, K, _zr2, 0)

    pltpu.sync_copy(den_v, denp_hbm.at[c, s])


def kernel(x, edge_index, W, att_src, att_dst, bias, ln_gamma, ln_beta):
    perm = jnp.array(_PERM, dtype=jnp.int32)
    Wp = W[:, perm]
    asp = att_src.reshape(1, C)[:, perm]
    adp = att_dst.reshape(1, C)[:, perm]
    h0, h1, a2, mx = _pre(x, Wp, asp, adp)
    asrc = a2[:, 0]
    adst = a2[:, 1]
    m0 = mx[0] + mx[1]
    msh = jnp.where(m0 >= 0, m0, 0.2 * m0)
    e2 = edge_index.reshape(2, NW, CHUNKS, K)
    outp, denp = _sc_edge(h0, h1, asrc, adst, msh, e2)
    dent = denp.reshape(NW, NPAD).transpose(1, 0)
    return _post(outp, dent, bias.reshape(1, C), ln_gamma.reshape(1, C),
                 ln_beta.reshape(1, C))


# speedup vs baseline: 1.5635x; 1.0555x over previous
import functools

import jax
import jax.numpy as jnp
from jax import lax
from jax.experimental import pallas as pl
from jax.experimental.pallas import tpu as pltpu
from jax.experimental.pallas import tpu_sc as plsc

N = 10000
E = 320000
C = 128
CH = C // 2
NC = 2
NS = 16
L = 16

NW = NC * NS
PT = E // NW
K = 80
CHUNKS = PT // K
NPAD = 10240
TROW = NPAD // NS
NB = 10
BN = NPAD // NB


_PERM = [32 * g + (k // 2 if k % 2 == 0 else 16 + k // 2)
         for g in range(4) for k in range(32)]


BA = N // 5


def _pre_body(x_ref, w_ref, att2_ref, h0_ref, h1_ref, a2_ref, mx_ref):
    h = jnp.dot(x_ref[...], w_ref[...], preferred_element_type=jnp.float32)
    hb = h.astype(jnp.bfloat16)
    h0_ref[...] = hb[:, :CH]
    h1_ref[...] = hb[:, CH:]
    a2 = jnp.dot(h, att2_ref[...], preferred_element_type=jnp.float32)
    a2_ref[...] = a2

    @pl.when(pl.program_id(0) == 0)
    def _():
        mx_ref[...] = jnp.full((2, C), -3.0e38, jnp.float32)

    blk = jnp.concatenate(
        [jnp.full((1, C), jnp.max(a2[:, 0]), jnp.float32),
         jnp.full((1, C), jnp.max(a2[:, 1]), jnp.float32)], axis=0)
    mx_ref[...] = jnp.maximum(mx_ref[...], blk)


def _pre(x, W, att2):
    return pl.pallas_call(
        _pre_body,
        grid=(5,),
        in_specs=[
            pl.BlockSpec((BA, C), lambda i: (i, 0)),
            pl.BlockSpec((C, C), lambda i: (0, 0)),
            pl.BlockSpec((C, 2), lambda i: (0, 0)),
        ],
        out_specs=[
            pl.BlockSpec((BA, CH), lambda i: (i, 0)),
            pl.BlockSpec((BA, CH), lambda i: (i, 0)),
            pl.BlockSpec((BA, 2), lambda i: (i, 0)),
            pl.BlockSpec((2, C), lambda i: (0, 0)),
        ],
        out_shape=[
            jax.ShapeDtypeStruct((N, CH), jnp.bfloat16),
            jax.ShapeDtypeStruct((N, CH), jnp.bfloat16),
            jax.ShapeDtypeStruct((N, 2), jnp.float32),
            jax.ShapeDtypeStruct((2, C), jnp.float32),
        ],
    )(x, W, att2)


def _post_body(p_ref, den_ref, b_ref, g_ref, bb_ref, y_ref):
    num = p_ref[0] + p_ref[1]
    cols = jnp.concatenate([num[0], num[1]], axis=1)
    den = jnp.sum(den_ref[...], axis=1)
    srow = cols / (den[:, None] + 1e-16) + b_ref[...]
    mu = jnp.mean(srow, axis=1, keepdims=True)
    var = jnp.mean((srow - mu) ** 2, axis=1, keepdims=True)
    yn = (srow - mu) * lax.rsqrt(var + 1e-5) * g_ref[...] + bb_ref[...]
    y_ref[...] = jnp.where(yn > 0, yn, jnp.exp(yn) - 1.0)


BR = N // 10


def _post(p, dent, bias, g, b):
    return pl.pallas_call(
        _post_body,
        grid=(10,),
        in_specs=[
            pl.BlockSpec((NC, 2, BR, CH), lambda i: (0, 0, i, 0)),
            pl.BlockSpec((BR, NW), lambda i: (i, 0)),
            pl.BlockSpec((1, C), lambda i: (0, 0)),
            pl.BlockSpec((1, C), lambda i: (0, 0)),
            pl.BlockSpec((1, C), lambda i: (0, 0)),
        ],
        out_specs=pl.BlockSpec((BR, C), lambda i: (i, 0)),
        out_shape=jax.ShapeDtypeStruct((N, C), jnp.float32),
    )(p, dent, bias, g, b)


@functools.partial(
    pl.kernel,
    out_type=[
        jax.ShapeDtypeStruct((NC, 2, NPAD, CH), jnp.float32),
        jax.ShapeDtypeStruct((NC, NS, NPAD), jnp.float32),
    ],
    mesh=plsc.VectorSubcoreMesh(core_axis_name="c", subcore_axis_name="s"),
    compiler_params=pltpu.CompilerParams(needs_layout_passes=False,
                                         use_tc_tiling_on_sc=False),
    scratch_types=[
        pltpu.VMEM((N,), jnp.float32),
        pltpu.VMEM((N,), jnp.float32),
        pltpu.VMEM((PT,), jnp.float32),
        pltpu.VMEM((NPAD,), jnp.float32),
        pltpu.VMEM((CHUNKS, K), jnp.int32),
        pltpu.VMEM((CHUNKS, K), jnp.int32),
        pltpu.VMEM((2, K, CH), jnp.bfloat16),
        pltpu.VMEM((2, K, CH), jnp.float32),
        pltpu.VMEM((C,), jnp.float32),
        pltpu.SemaphoreType.DMA,
        pltpu.SemaphoreType.DMA,
        pltpu.SemaphoreType.DMA,
        pltpu.SemaphoreType.DMA,
        pltpu.VMEM_SHARED((NPAD, CH), jnp.float32),
    ],
)
def _sc_edge(h0_hbm, h1_hbm, asrc_hbm, adst_hbm, msh_hbm, e2_hbm,
             outp_hbm, denp_hbm,
             asrc_v, adst_v, e_v, den_v, src_v, dst_v, rowsb_v, rowsf_v,
             msh_v, gsem0, gsem1, ssem0, ssem1, sh_out):
    c = lax.axis_index("c")
    s = lax.axis_index("s")
    wid = c * NS + s

    zf = jnp.zeros((L,), jnp.float32)

    pltpu.sync_copy(asrc_hbm, asrc_v)
    pltpu.sync_copy(adst_hbm, adst_v)
    pltpu.sync_copy(msh_hbm, msh_v)
    pltpu.sync_copy(e2_hbm.at[0, wid], src_v)
    pltpu.sync_copy(e2_hbm.at[1, wid], dst_v)

    def _zrow(i, _):
        for v in range(CH // L):
            rowsf_v[0, i, pl.ds(v * L, L)] = zf
        return 0

    lax.fori_loop(0, K, _zrow, 0)

    def _zden(i, _):
        den_v[pl.ds(i * L, L)] = zf
        return 0

    lax.fori_loop(0, NPAD // L, _zden, 0)

    m_shift = msh_v[pl.ds(0, L)]
    row_base = pl.multiple_of(s * TROW, 8)

    for half in range(2):
        h_hbm = (h0_hbm, h1_hbm)[half]

        for k in range(TROW // K):
            pltpu.sync_copy(rowsf_v.at[0],
                            sh_out.at[pl.ds(row_base + k * K, K)])
        plsc.subcore_barrier()

        def _gather(i, gb, h_hbm=h_hbm):
            sem = (gsem0, gsem1)[gb]
            return pltpu.make_async_copy(h_hbm.at[src_v.at[i]],
                                         rowsb_v.at[gb], sem)

        def _scat_start(i, fb):
            pltpu.async_copy(rowsf_v.at[fb], sh_out.at[dst_v.at[i]],
                             (ssem0, ssem1)[fb], add=True)

        def _scat_wait(i, fb):
            pltpu.make_async_copy(rowsf_v.at[fb], sh_out.at[dst_v.at[i]],
                                  (ssem0, ssem1)[fb]).wait()

        def _att(i):
            for g in range(K // L):
                si = src_v[i, pl.ds(g * L, L)]
                di = dst_v[i, pl.ds(g * L, L)]
                al = (plsc.load_gather(asrc_v, [si])
                      + plsc.load_gather(adst_v, [di]))
                al = jnp.where(al >= 0, al, 0.2 * al)
                e = jnp.exp(al - m_shift)
                e_v[pl.ds(i * K + g * L, L)] = e
                plsc.addupdate_scatter(den_v, [di], e)

        def _scale(i, gb, fb):
            for g in range(K // L):
                ev = e_v[pl.ds(i * K + g * L, L)]
                for j in range(L):
                    aj = jnp.broadcast_to(ev[j], (L,))
                    r = g * L + j
                    for v in range(CH // 32):
                        x = rowsb_v[gb, r, pl.ds(32 * v, 32)]
                        u, w = plsc.unpack(
                            x, format=plsc.PackFormat.INTERLEAVED)
                        rowsf_v[fb, r, pl.ds(32 * v, L)] = u * aj
                        rowsf_v[fb, r, pl.ds(32 * v + L, L)] = w * aj

        def _step(i, b, first, ahead=True, half=half):
            if not first:
                _scat_wait(i - 2, b)
            if ahead:
                _gather(i + 1, 1 - b).start()
            if half == 0:
                _att(i)
            _gather(i, b).wait()
            _scale(i, b, b)
            _scat_start(i, b)

        _gather(0, 0).start()
        _step(0, 0, True)
        _step(1, 1, True)
        _step(2, 0, False)

        def _two(t, _):
            i0 = 3 + t * 2
            _step(i0, 1, False)
            _step(i0 + 1, 0, False)
            return 0

        lax.fori_loop(0, (CHUNKS - 5) // 2, _two, 0)
        for i in range(CHUNKS - 2, CHUNKS):
            _step(i, i % 2, False, ahead=(i + 1 < CHUNKS))
        _scat_wait(CHUNKS - 2, (CHUNKS - 2) % 2)
        _scat_wait(CHUNKS - 1, (CHUNKS - 1) % 2)

        plsc.subcore_barrier()
        pltpu.sync_copy(sh_out.at[pl.ds(row_base, TROW)],
                        outp_hbm.at[c, half, pl.ds(row_base, TROW), :])

        def _zr2(i, _):
            for v in range(CH // L):
                rowsf_v[0, i, pl.ds(v * L, L)] = zf
            return 0

        if half == 0:
            lax.fori_loop(0, K, _zr2, 0)

    pltpu.sync_copy(den_v, denp_hbm.at[c, s])


def kernel(x, edge_index, W, att_src, att_dst, bias, ln_gamma, ln_beta):
    perm = jnp.array(_PERM, dtype=jnp.int32)
    Wp = W[:, perm]
    att2 = jnp.concatenate([att_src.reshape(C, 1), att_dst.reshape(C, 1)],
                           axis=1)[perm, :]
    h0, h1, a2, mx = _pre(x, Wp, att2)
    asrc = a2[:, 0]
    adst = a2[:, 1]
    m0 = mx[0] + mx[1]
    msh = jnp.where(m0 >= 0, m0, 0.2 * m0)
    e2 = edge_index.reshape(2, NW, CHUNKS, K)
    outp, denp = _sc_edge(h0, h1, asrc, adst, msh, e2)
    dent = denp.reshape(NW, NPAD).transpose(1, 0)
    return _post(outp, dent, bias.reshape(1, C), ln_gamma.reshape(1, C),
                 ln_beta.reshape(1, C))


# speedup vs baseline: 1.5650x; 1.0010x over previous
import functools

import jax
import jax.numpy as jnp
from jax import lax
from jax.experimental import pallas as pl
from jax.experimental.pallas import tpu as pltpu
from jax.experimental.pallas import tpu_sc as plsc

N = 10000
E = 320000
C = 128
CH = C // 2
NC = 2
NS = 16
L = 16

NW = NC * NS
PT = E // NW
K = 80
CHUNKS = PT // K
NPAD = 10240
TROW = NPAD // NS


_PERM = [32 * g + (k // 2 if k % 2 == 0 else 16 + k // 2)
         for g in range(4) for k in range(32)]


BA = N // 5


def _pre_body(x_ref, w_ref, att2_ref, h0_ref, h1_ref, a2_ref, mx_ref):
    h = jnp.dot(x_ref[...], w_ref[...], preferred_element_type=jnp.float32)
    hb = h.astype(jnp.bfloat16)
    h0_ref[...] = hb[:, :CH]
    h1_ref[...] = hb[:, CH:]
    a2 = jnp.dot(h, att2_ref[...], preferred_element_type=jnp.float32)
    a2_ref[...] = a2

    @pl.when(pl.program_id(0) == 0)
    def _():
        mx_ref[...] = jnp.full((2, C), -3.0e38, jnp.float32)

    blk = jnp.concatenate(
        [jnp.full((1, C), jnp.max(a2[:, 0]), jnp.float32),
         jnp.full((1, C), jnp.max(a2[:, 1]), jnp.float32)], axis=0)
    mx_ref[...] = jnp.maximum(mx_ref[...], blk)


def _pre(x, W, att2):
    return pl.pallas_call(
        _pre_body,
        grid=(5,),
        in_specs=[
            pl.BlockSpec((BA, C), lambda i: (i, 0)),
            pl.BlockSpec((C, C), lambda i: (0, 0)),
            pl.BlockSpec((C, 2), lambda i: (0, 0)),
        ],
        out_specs=[
            pl.BlockSpec((BA, CH), lambda i: (i, 0)),
            pl.BlockSpec((BA, CH), lambda i: (i, 0)),
            pl.BlockSpec((BA, 2), lambda i: (i, 0)),
            pl.BlockSpec((2, C), lambda i: (0, 0)),
        ],
        out_shape=[
            jax.ShapeDtypeStruct((N, CH), jnp.bfloat16),
            jax.ShapeDtypeStruct((N, CH), jnp.bfloat16),
            jax.ShapeDtypeStruct((N, 2), jnp.float32),
            jax.ShapeDtypeStruct((2, C), jnp.float32),
        ],
    )(x, W, att2)


def _post_body(p_ref, den_ref, b_ref, g_ref, bb_ref, y_ref):
    num = p_ref[0] + p_ref[1]
    cols = jnp.concatenate([num[0], num[1]], axis=1)
    den = jnp.sum(den_ref[...], axis=1)
    srow = cols / (den[:, None] + 1e-16) + b_ref[...]
    mu = jnp.mean(srow, axis=1, keepdims=True)
    var = jnp.mean((srow - mu) ** 2, axis=1, keepdims=True)
    yn = (srow - mu) * lax.rsqrt(var + 1e-5) * g_ref[...] + bb_ref[...]
    y_ref[...] = jnp.where(yn > 0, yn, jnp.exp(yn) - 1.0)


BR = N // 10


def _post(p, dent, bias, g, b):
    return pl.pallas_call(
        _post_body,
        grid=(10,),
        in_specs=[
            pl.BlockSpec((NC, 2, BR, CH), lambda i: (0, 0, i, 0)),
            pl.BlockSpec((BR, NW), lambda i: (i, 0)),
            pl.BlockSpec((1, C), lambda i: (0, 0)),
            pl.BlockSpec((1, C), lambda i: (0, 0)),
            pl.BlockSpec((1, C), lambda i: (0, 0)),
        ],
        out_specs=pl.BlockSpec((BR, C), lambda i: (i, 0)),
        out_shape=jax.ShapeDtypeStruct((N, C), jnp.float32),
    )(p, dent, bias, g, b)


@functools.partial(
    pl.kernel,
    out_type=[
        jax.ShapeDtypeStruct((NC, 2, NPAD, CH), jnp.float32),
        jax.ShapeDtypeStruct((NC, NS, NPAD), jnp.float32),
    ],
    mesh=plsc.VectorSubcoreMesh(core_axis_name="c", subcore_axis_name="s"),
    compiler_params=pltpu.CompilerParams(needs_layout_passes=False,
                                         use_tc_tiling_on_sc=False),
    scratch_types=[
        pltpu.VMEM((N,), jnp.float32),
        pltpu.VMEM((N,), jnp.float32),
        pltpu.VMEM((PT,), jnp.float32),
        pltpu.VMEM((NPAD,), jnp.float32),
        pltpu.VMEM((CHUNKS, K), jnp.int32),
        pltpu.VMEM((CHUNKS, K), jnp.int32),
        pltpu.VMEM((2, K, CH), jnp.bfloat16),
        pltpu.VMEM((2, K, CH), jnp.float32),
        pltpu.VMEM((C,), jnp.float32),
        pltpu.SemaphoreType.DMA,
        pltpu.SemaphoreType.DMA,
        pltpu.SemaphoreType.DMA,
        pltpu.SemaphoreType.DMA,
        pltpu.VMEM_SHARED((NPAD, CH), jnp.float32),
    ],
)
def _sc_edge(h0_hbm, h1_hbm, asrc_hbm, adst_hbm, msh_hbm, e2_hbm,
             outp_hbm, denp_hbm,
             asrc_v, adst_v, e_v, den_v, src_v, dst_v, rowsb_v, rowsf_v,
             msh_v, gsem0, gsem1, ssem0, ssem1, sh_out):
    c = lax.axis_index("c")
    s = lax.axis_index("s")
    wid = c * NS + s

    zf = jnp.zeros((L,), jnp.float32)

    pltpu.sync_copy(asrc_hbm, asrc_v)
    pltpu.sync_copy(adst_hbm, adst_v)
    pltpu.sync_copy(msh_hbm, msh_v)
    pltpu.sync_copy(e2_hbm.at[0, wid], src_v)
    pltpu.sync_copy(e2_hbm.at[1, wid], dst_v)

    def _zrow(i, _):
        for v in range(CH // L):
            rowsf_v[0, i, pl.ds(v * L, L)] = zf
        return 0

    lax.fori_loop(0, K, _zrow, 0)

    def _zden(i, _):
        den_v[pl.ds(i * L, L)] = zf
        return 0

    lax.fori_loop(0, NPAD // L, _zden, 0)

    m_shift = msh_v[pl.ds(0, L)]
    row_base = pl.multiple_of(s * TROW, 8)

    for half in range(2):
        h_hbm = (h0_hbm, h1_hbm)[half]

        for k in range(TROW // K):
            pltpu.sync_copy(rowsf_v.at[0],
                            sh_out.at[pl.ds(row_base + k * K, K)])
        plsc.subcore_barrier()

        def _gather(i, gb, h_hbm=h_hbm):
            sem = (gsem0, gsem1)[gb]
            return pltpu.make_async_copy(h_hbm.at[src_v.at[i]],
                                         rowsb_v.at[gb], sem)

        def _scat_start(i, fb):
            pltpu.async_copy(rowsf_v.at[fb], sh_out.at[dst_v.at[i]],
                             (ssem0, ssem1)[fb], add=True)

        def _scat_wait(i, fb):
            pltpu.make_async_copy(rowsf_v.at[fb], sh_out.at[dst_v.at[i]],
                                  (ssem0, ssem1)[fb]).wait()

        def _att(i):
            for g in range(K // L):
                si = src_v[i, pl.ds(g * L, L)]
                di = dst_v[i, pl.ds(g * L, L)]
                al = (plsc.load_gather(asrc_v, [si])
                      + plsc.load_gather(adst_v, [di]))
                al = jnp.where(al >= 0, al, 0.2 * al)
                e = jnp.exp(al - m_shift)
                e_v[pl.ds(i * K + g * L, L)] = e
                plsc.addupdate_scatter(den_v, [di], e)

        def _scale(i, gb, fb):
            for g in range(K // L):
                ev = e_v[pl.ds(i * K + g * L, L)]
                for j in range(L):
                    aj = jnp.broadcast_to(ev[j], (L,))
                    r = g * L + j
                    for v in range(CH // 32):
                        x = rowsb_v[gb, r, pl.ds(32 * v, 32)]
                        u, w = plsc.unpack(
                            x, format=plsc.PackFormat.INTERLEAVED)
                        rowsf_v[fb, r, pl.ds(32 * v, L)] = u * aj
                        rowsf_v[fb, r, pl.ds(32 * v + L, L)] = w * aj

        def _step(i, b, first, ahead=True, half=half):
            if not first:
                _scat_wait(i - 2, b)
            if ahead:
                _gather(i + 1, 1 - b).start()
            if half == 0:
                _att(i)
            _gather(i, b).wait()
            _scale(i, b, b)
            _scat_start(i, b)

        _gather(0, 0).start()
        _step(0, 0, True)
        _step(1, 1, True)
        _step(2, 0, False)

        def _two(t, _):
            i0 = 3 + t * 2
            _step(i0, 1, False)
            _step(i0 + 1, 0, False)
            return 0

        lax.fori_loop(0, (CHUNKS - 5) // 2, _two, 0)
        for i in range(CHUNKS - 2, CHUNKS):
            _step(i, i % 2, False, ahead=(i + 1 < CHUNKS))
        _scat_wait(CHUNKS - 2, (CHUNKS - 2) % 2)
        _scat_wait(CHUNKS - 1, (CHUNKS - 1) % 2)

        plsc.subcore_barrier()
        pltpu.sync_copy(sh_out.at[pl.ds(row_base, TROW)],
                        outp_hbm.at[c, half, pl.ds(row_base, TROW), :])

        def _zr2(i, _):
            for v in range(CH // L):
                rowsf_v[0, i, pl.ds(v * L, L)] = zf
            return 0

        if half == 0:
            lax.fori_loop(0, K, _zr2, 0)

    pltpu.sync_copy(den_v, denp_hbm.at[c, s])


def kernel(x, edge_index, W, att_src, att_dst, bias, ln_gamma, ln_beta):
    perm = jnp.array(_PERM, dtype=jnp.int32)
    Wp = W[:, perm]
    att2 = jnp.concatenate([att_src.reshape(C, 1), att_dst.reshape(C, 1)],
                           axis=1)[perm, :]
    h0, h1, a2, mx = _pre(x, Wp, att2)
    asrc = a2[:, 0]
    adst = a2[:, 1]
    m0 = mx[0] + mx[1]
    msh = jnp.where(m0 >= 0, m0, 0.2 * m0)
    e2 = edge_index.reshape(2, NW, CHUNKS, K)
    outp, denp = _sc_edge(h0, h1, asrc, adst, msh, e2)
    dent = denp.reshape(NW, NPAD).transpose(1, 0)
    return _post(outp, dent, bias.reshape(1, C), ln_gamma.reshape(1, C),
                 ln_beta.reshape(1, C))
